# baseline (speedup 1.0000x reference)
import functools

import jax
import jax.numpy as jnp
from jax import lax
from jax.experimental import pallas as pl
from jax.experimental.pallas import tpu as pltpu
from jax.experimental.pallas import tpu_sc as plsc

NUM_CORES = 2
NUM_SUBCORES = 16
NUM_WORKERS = NUM_CORES * NUM_SUBCORES
LANES = 16

BATCH = 16384
EMBED_DIM = 64
ROWS_PER_WORKER = BATCH // NUM_WORKERS
CHUNK = 128
NUM_CHUNKS = ROWS_PER_WORKER // CHUNK
GROUPS = ROWS_PER_WORKER // LANES


def _bpr_body(users_hbm, items_hbm, utab_hbm, itab_hbm, out_hbm,
              idx_u, idx_i, rows_u, rows_i, out_v, scratch_flat, sem):
    wid = lax.axis_index("s") * NUM_CORES + lax.axis_index("c")

    base = wid * NUM_CHUNKS
    pltpu.sync_copy(users_hbm.at[pl.ds(base, NUM_CHUNKS)], idx_u)
    pltpu.sync_copy(items_hbm.at[pl.ds(base, NUM_CHUNKS)], idx_i)

    copies = []
    for c in range(NUM_CHUNKS):
        copies.append(pltpu.async_copy(
            utab_hbm.at[idx_u.at[c]], rows_u.at[pl.ds(c * CHUNK, CHUNK)], sem))
        copies.append(pltpu.async_copy(
            itab_hbm.at[idx_i.at[c]], rows_i.at[pl.ds(c * CHUNK, CHUNK)], sem))
    for cp in copies:
        cp.wait()

    iota = lax.iota(jnp.int32, LANES)

    def group(g, _):
        for k in range(LANES):
            r = g * LANES + k
            acc = rows_u[r, pl.ds(0, LANES)] * rows_i[r, pl.ds(0, LANES)]
            for c in range(1, EMBED_DIM // LANES):
                acc = acc + (rows_u[r, pl.ds(c * LANES, LANES)]
                             * rows_i[r, pl.ds(c * LANES, LANES)])
            scratch_flat[pl.ds(k * LANES, LANES)] = acc
        iota16 = iota * LANES
        res = plsc.load_gather(scratch_flat, [iota16])
        for j in range(1, LANES):
            res = res + plsc.load_gather(scratch_flat, [iota16 + j])
        out_v[pl.ds(pl.multiple_of(g * LANES, LANES), LANES)] = res
        return 0

    lax.fori_loop(0, GROUPS, group, 0)

    pltpu.sync_copy(out_v, out_hbm.at[pl.ds(wid * ROWS_PER_WORKER,
                                            ROWS_PER_WORKER)])


@jax.jit
def _bpr_sc(users2d, items2d, user_table, item_table):
    mesh = plsc.VectorSubcoreMesh(
        core_axis_name="c", subcore_axis_name="s",
        num_cores=NUM_CORES, num_subcores=NUM_SUBCORES)
    return pl.kernel(
        _bpr_body,
        out_type=jax.ShapeDtypeStruct((BATCH,), jnp.float32),
        mesh=mesh,
        compiler_params=pltpu.CompilerParams(
            needs_layout_passes=False, use_tc_tiling_on_sc=False),
        scratch_types=[
            pltpu.VMEM((NUM_CHUNKS, CHUNK), jnp.int32),
            pltpu.VMEM((NUM_CHUNKS, CHUNK), jnp.int32),
            pltpu.VMEM((ROWS_PER_WORKER, EMBED_DIM), jnp.float32),
            pltpu.VMEM((ROWS_PER_WORKER, EMBED_DIM), jnp.float32),
            pltpu.VMEM((ROWS_PER_WORKER,), jnp.float32),
            pltpu.VMEM((LANES * LANES,), jnp.float32),
            pltpu.SemaphoreType.DMA,
        ],
    )(users2d, items2d, user_table, item_table)


def kernel(users, items, user_table, item_table):
    users2d = users.astype(jnp.int32).reshape(NUM_WORKERS * NUM_CHUNKS, CHUNK)
    items2d = items.astype(jnp.int32).reshape(NUM_WORKERS * NUM_CHUNKS, CHUNK)
    return _bpr_sc(users2d, items2d, user_table, item_table)
